# Initial kernel scaffold; baseline (speedup 1.0000x reference)
import functools

import jax
import jax.numpy as jnp
from jax import lax
from jax.experimental import pallas as pl
from jax.experimental.pallas import tpu as pltpu
from jax.experimental.pallas import tpu_sc as plsc

_NUM_CORES = 2
_NUM_SUBCORES = 16
_NW = _NUM_CORES * _NUM_SUBCORES
_LANES = 16


@functools.lru_cache(maxsize=None)
def _make_sc_kernel(rows, d, seq, rpw):
    ch = seq
    nchunk = rpw // ch
    mesh = plsc.VectorSubcoreMesh(
        core_axis_name="c", subcore_axis_name="s",
        num_cores=_NUM_CORES, num_subcores=_NUM_SUBCORES)

    @functools.partial(
        pl.kernel,
        mesh=mesh,
        out_type=jax.ShapeDtypeStruct((rows, d), jnp.float32),
        scratch_types=[
            pltpu.VMEM((rpw,), jnp.int32),
            pltpu.VMEM((ch, d), jnp.float32),
            pltpu.VMEM((ch, d), jnp.float32),
            pltpu.SemaphoreType.DMA,
        ],
    )
    def k(table_hbm, idx_hbm, pos_hbm, out_hbm, idx_v, rows_v, pos_v, sem):
        wid = lax.axis_index("s") * _NUM_CORES + lax.axis_index("c")
        base = wid * rpw
        pltpu.sync_copy(idx_hbm.at[pl.ds(base, rpw)], idx_v)
        pltpu.sync_copy(pos_hbm, pos_v)

        def chunk_body(j, carry):
            off = j * ch
            pltpu.async_copy(
                table_hbm.at[idx_v.at[pl.ds(off, ch)]], rows_v, sem).wait()

            def add_body(r, c2):
                for cc in range(d // _LANES):
                    sl = pl.ds(cc * _LANES, _LANES)
                    plsc.addupdate(rows_v.at[r, sl], pos_v[r, sl])
                return c2

            lax.fori_loop(0, ch, add_body, 0, unroll=2)
            pltpu.sync_copy(rows_v, out_hbm.at[pl.ds(base + off, ch)])
            return carry

        lax.fori_loop(0, nchunk, chunk_body, 0)

    return k


def kernel(response_sequence, response_table, positional_table):
    b, s = response_sequence.shape
    v, d = response_table.shape
    rows = b * s
    rpw = rows // _NW
    idx = response_sequence.reshape(rows)
    k = _make_sc_kernel(rows, d, s, rpw)
    out = k(response_table, idx, positional_table)
    return out.reshape(b, s, d)


# speedup vs baseline: 2.4576x; 2.4576x over previous
import functools

import jax
import jax.numpy as jnp
from jax import lax
from jax.experimental import pallas as pl
from jax.experimental.pallas import tpu as pltpu
from jax.experimental.pallas import tpu_sc as plsc

_NUM_CORES = 2
_NUM_SUBCORES = 16
_NW = _NUM_CORES * _NUM_SUBCORES
_LANES = 16


@functools.lru_cache(maxsize=None)
def _make_sc_kernel(rows, d, seq, rpw):
    ch = seq
    nchunk = rpw // ch
    mesh = plsc.VectorSubcoreMesh(
        core_axis_name="c", subcore_axis_name="s",
        num_cores=_NUM_CORES, num_subcores=_NUM_SUBCORES)

    @functools.partial(
        pl.kernel,
        mesh=mesh,
        out_type=jax.ShapeDtypeStruct((rows, d), jnp.float32),
        scratch_types=[
            pltpu.VMEM((rpw,), jnp.int32),
            pltpu.VMEM((ch, d), jnp.float32),
            pltpu.VMEM((ch, d), jnp.float32),
            pltpu.SemaphoreType.DMA,
        ],
        compiler_params=pltpu.CompilerParams(use_tc_tiling_on_sc=False),
    )
    def k(table_hbm, idx_hbm, pos_hbm, out_hbm, idx_v, rows_v, pos_v, sem):
        wid = lax.axis_index("s") * _NUM_CORES + lax.axis_index("c")
        base = wid * rpw
        pltpu.sync_copy(idx_hbm.at[pl.ds(base, rpw)], idx_v)
        pltpu.sync_copy(pos_hbm, pos_v)

        def chunk_body(j, carry):
            off = j * ch
            pltpu.async_copy(
                table_hbm.at[idx_v.at[pl.ds(off, ch)]], rows_v, sem).wait()

            def add_body(r, c2):
                for cc in range(d // _LANES):
                    sl = pl.ds(cc * _LANES, _LANES)
                    plsc.addupdate(rows_v.at[r, sl], pos_v[r, sl])
                return c2

            lax.fori_loop(0, ch, add_body, 0, unroll=2)
            pltpu.sync_copy(rows_v, out_hbm.at[pl.ds(base + off, ch)])
            return carry

        lax.fori_loop(0, nchunk, chunk_body, 0)

    return k


def kernel(response_sequence, response_table, positional_table):
    b, s = response_sequence.shape
    v, d = response_table.shape
    rows = b * s
    rpw = rows // _NW
    idx = response_sequence.reshape(rows)
    k = _make_sc_kernel(rows, d, s, rpw)
    out = k(response_table, idx, positional_table)
    return out.reshape(b, s, d)


# speedup vs baseline: 2.8576x; 1.1628x over previous
import functools

import jax
import jax.numpy as jnp
from jax import lax
from jax.experimental import pallas as pl
from jax.experimental.pallas import tpu as pltpu
from jax.experimental.pallas import tpu_sc as plsc

_NUM_CORES = 2
_NUM_SUBCORES = 16
_NW = _NUM_CORES * _NUM_SUBCORES
_LANES = 16
_NBUF = 4


@functools.lru_cache(maxsize=None)
def _make_sc_kernel(rows, d, seq, rpw):
    ch = seq
    nchunk = rpw // ch
    nb = _NBUF
    nround = nchunk // nb
    assert nchunk % nb == 0
    mesh = plsc.VectorSubcoreMesh(
        core_axis_name="c", subcore_axis_name="s",
        num_cores=_NUM_CORES, num_subcores=_NUM_SUBCORES)

    @functools.partial(
        pl.kernel,
        mesh=mesh,
        out_type=jax.ShapeDtypeStruct((rows, d), jnp.float32),
        scratch_types=[
            pltpu.VMEM((rpw,), jnp.int32),
            pltpu.VMEM((nb, ch, d), jnp.float32),
            pltpu.VMEM((ch, d), jnp.float32),
            pltpu.SemaphoreType.DMA((nb,)),
            pltpu.SemaphoreType.DMA((nb,)),
        ],
        compiler_params=pltpu.CompilerParams(use_tc_tiling_on_sc=False),
    )
    def k(table_hbm, idx_hbm, pos_hbm, out_hbm, idx_v, rows_v, pos_v, sg, so):
        wid = lax.axis_index("s") * _NUM_CORES + lax.axis_index("c")
        base = wid * rpw
        pltpu.sync_copy(idx_hbm.at[pl.ds(base, rpw)], idx_v)
        pltpu.sync_copy(pos_hbm, pos_v)

        def gather(j, b):
            pltpu.async_copy(
                table_hbm.at[idx_v.at[pl.ds(j * ch, ch)]],
                rows_v.at[b], sg.at[b])

        def wait_gather(b):
            pltpu.make_async_copy(
                table_hbm.at[idx_v.at[pl.ds(0, ch)]],
                rows_v.at[b], sg.at[b]).wait()

        def wait_out(b):
            pltpu.make_async_copy(
                rows_v.at[b], out_hbm.at[pl.ds(0, ch)], so.at[b]).wait()

        for p in range(nb - 1):
            gather(p, p)

        def round_body(g, carry):
            j0 = g * nb
            for b in range(nb):
                j = j0 + b
                wait_gather(b)

                def add_body(r, c2, _b=b):
                    for cc in range(d // _LANES):
                        sl = pl.ds(cc * _LANES, _LANES)
                        plsc.addupdate(rows_v.at[_b, r, sl], pos_v[r, sl])
                    return c2

                lax.fori_loop(0, ch, add_body, 0, unroll=4)
                pltpu.async_copy(
                    rows_v.at[b], out_hbm.at[pl.ds(base + j * ch, ch)],
                    so.at[b])

                jg = j + nb - 1
                bg = (b - 1) % nb

                @pl.when(jnp.logical_and(jg < nchunk, j >= 1))
                def _():
                    wait_out(bg)

                @pl.when(jg < nchunk)
                def _():
                    gather(jg, bg)
            return carry

        lax.fori_loop(0, nround, round_body, 0)
        for b in range(nb):
            wait_out(b)

    return k


def kernel(response_sequence, response_table, positional_table):
    b, s = response_sequence.shape
    v, d = response_table.shape
    rows = b * s
    rpw = rows // _NW
    idx = response_sequence.reshape(rows)
    k = _make_sc_kernel(rows, d, s, rpw)
    out = k(response_table, idx, positional_table)
    return out.reshape(b, s, d)


# speedup vs baseline: 2.8580x; 1.0001x over previous
import functools

import jax
import jax.numpy as jnp
from jax import lax
from jax.experimental import pallas as pl
from jax.experimental.pallas import tpu as pltpu
from jax.experimental.pallas import tpu_sc as plsc

_NUM_CORES = 2
_NUM_SUBCORES = 16
_NW = _NUM_CORES * _NUM_SUBCORES
_LANES = 16
_NBUF = 4


@functools.lru_cache(maxsize=None)
def _make_sc_kernel(batch, seq, d):
    ch = seq
    bpw = batch // _NW
    nb = _NBUF
    nround = bpw // nb
    assert bpw % nb == 0
    mesh = plsc.VectorSubcoreMesh(
        core_axis_name="c", subcore_axis_name="s",
        num_cores=_NUM_CORES, num_subcores=_NUM_SUBCORES)

    @functools.partial(
        pl.kernel,
        mesh=mesh,
        out_type=jax.ShapeDtypeStruct((batch, seq, d), jnp.float32),
        scratch_types=[
            pltpu.VMEM((bpw, ch), jnp.int32),
            pltpu.VMEM((nb, ch, d), jnp.float32),
            pltpu.VMEM((ch, d), jnp.float32),
            pltpu.SemaphoreType.DMA((nb,)),
            pltpu.SemaphoreType.DMA((nb,)),
        ],
        compiler_params=pltpu.CompilerParams(use_tc_tiling_on_sc=False),
    )
    def k(table_hbm, idx_hbm, pos_hbm, out_hbm, idx_v, rows_v, pos_v, sg, so):
        wid = lax.axis_index("s") * _NUM_CORES + lax.axis_index("c")
        b0 = wid * bpw
        pltpu.sync_copy(idx_hbm.at[pl.ds(b0, bpw), :], idx_v)
        pltpu.sync_copy(pos_hbm, pos_v)

        def gather(j, b):
            pltpu.async_copy(
                table_hbm.at[idx_v.at[j]], rows_v.at[b], sg.at[b])

        def wait_gather(b):
            pltpu.make_async_copy(
                table_hbm.at[idx_v.at[0]], rows_v.at[b], sg.at[b]).wait()

        def wait_out(b):
            pltpu.make_async_copy(
                rows_v.at[b], out_hbm.at[0], so.at[b]).wait()

        for p in range(nb - 1):
            gather(p, p)

        def round_body(g, carry):
            j0 = g * nb
            for b in range(nb):
                j = j0 + b
                wait_gather(b)

                def add_body(r, c2, _b=b):
                    for cc in range(d // _LANES):
                        sl = pl.ds(cc * _LANES, _LANES)
                        plsc.addupdate(rows_v.at[_b, r, sl], pos_v[r, sl])
                    return c2

                lax.fori_loop(0, ch, add_body, 0, unroll=4)
                pltpu.async_copy(rows_v.at[b], out_hbm.at[b0 + j], so.at[b])

                jg = j + nb - 1
                bg = (b - 1) % nb

                @pl.when(jnp.logical_and(jg < bpw, j >= 1))
                def _():
                    wait_out(bg)

                @pl.when(jg < bpw)
                def _():
                    gather(jg, bg)
            return carry

        lax.fori_loop(0, nround, round_body, 0)
        for b in range(nb):
            wait_out(b)

    return k


def kernel(response_sequence, response_table, positional_table):
    b, s = response_sequence.shape
    v, d = response_table.shape
    k = _make_sc_kernel(b, s, d)
    return k(response_table, response_sequence, positional_table)
